# Initial kernel scaffold; baseline (speedup 1.0000x reference)
import jax
import jax.numpy as jnp
from jax.experimental import pallas as pl


def kernel(pos_u, pos_v_l, pos_v_r, neg_v_l, neg_v_r, u_l_weight, u_r_weight, v_l_weight, v_r_weight):
    raise NotImplementedError("write your pallas kernel here")


# speedup vs baseline: 4.1471x; 4.1471x over previous
import functools

import jax
import jax.numpy as jnp
from jax import lax
from jax.experimental import pallas as pl
from jax.experimental.pallas import tpu as pltpu
from jax.experimental.pallas import tpu_sc as plsc

EMB_DIM = 64
BATCH = 16384
NEG = 20
NTILES = 32
ITEMS_PER_TILE = BATCH // NTILES
SUB = 16
NSUB = ITEMS_PER_TILE // SUB
NEG_ROWS = SUB * NEG
NEG_PARTS = 4
PART = NEG_ROWS // NEG_PARTS
OUT_COLS = 48


def _sc_scores(u_l, u_r, v_l, v_r, pu2, pvl2, pvr2, nl2, nr2):
    mesh = plsc.VectorSubcoreMesh(core_axis_name="c", subcore_axis_name="s")

    @functools.partial(
        pl.kernel,
        out_type=jax.ShapeDtypeStruct((BATCH // SUB, OUT_COLS, SUB), jnp.float32),
        mesh=mesh,
        compiler_params=pltpu.CompilerParams(
            needs_layout_passes=False, use_tc_tiling_on_sc=False),
        scratch_types=[
            pltpu.VMEM((NSUB, SUB), jnp.int32),
            pltpu.VMEM((NSUB, SUB), jnp.int32),
            pltpu.VMEM((NSUB, SUB), jnp.int32),
            pltpu.VMEM((NSUB * NEG_PARTS, PART), jnp.int32),
            pltpu.VMEM((NSUB * NEG_PARTS, PART), jnp.int32),
            pltpu.VMEM((SUB, EMB_DIM), jnp.float32),
            pltpu.VMEM((SUB, EMB_DIM), jnp.float32),
            pltpu.VMEM((SUB, EMB_DIM), jnp.float32),
            pltpu.VMEM((SUB, EMB_DIM), jnp.float32),
            pltpu.VMEM((NEG_ROWS, EMB_DIM), jnp.float32),
            pltpu.VMEM((NEG_ROWS, EMB_DIM), jnp.float32),
            pltpu.VMEM((OUT_COLS, SUB), jnp.float32),
            pltpu.SemaphoreType.DMA,
        ],
    )
    def k(u_l_h, u_r_h, v_l_h, v_r_h, pu_h, pvl_h, pvr_h, nl_h, nr_h, out_h,
          pu_v, pvl_v, pvr_v, nl_v, nr_v,
          eul, eur, evl, evr, enl, enr, sco, sem):
        wid = lax.axis_index("s") * 2 + lax.axis_index("c")

        pltpu.sync_copy(pu_h.at[pl.ds(wid * NSUB, NSUB)], pu_v)
        pltpu.sync_copy(pvl_h.at[pl.ds(wid * NSUB, NSUB)], pvl_v)
        pltpu.sync_copy(pvr_h.at[pl.ds(wid * NSUB, NSUB)], pvr_v)
        nrows = NSUB * NEG_PARTS
        pltpu.sync_copy(nl_h.at[pl.ds(wid * nrows, nrows)], nl_v)
        pltpu.sync_copy(nr_h.at[pl.ds(wid * nrows, nrows)], nr_v)

        lane = lax.iota(jnp.int32, 16)
        lane20 = lane * NEG
        zeros = jnp.zeros((16,), jnp.float32)
        for c in range(2 + 2 * NEG, OUT_COLS):
            sco[c, :] = zeros

        def sub_body(j, carry):
            cps = [
                pltpu.async_copy(u_l_h.at[pu_v.at[j]], eul, sem),
                pltpu.async_copy(u_r_h.at[pu_v.at[j]], eur, sem),
                pltpu.async_copy(v_l_h.at[pvl_v.at[j]], evl, sem),
                pltpu.async_copy(v_r_h.at[pvr_v.at[j]], evr, sem),
            ]
            for p in range(NEG_PARTS):
                cps.append(pltpu.async_copy(
                    v_l_h.at[nl_v.at[j * NEG_PARTS + p]],
                    enl.at[pl.ds(p * PART, PART)], sem))
                cps.append(pltpu.async_copy(
                    v_r_h.at[nr_v.at[j * NEG_PARTS + p]],
                    enr.at[pl.ds(p * PART, PART)], sem))
            for cp in cps:
                cp.wait()

            def side(eu, ev, en):
                def dbody(dd, acc):
                    pos, negs = acc
                    dvec = jnp.full((16,), dd, jnp.int32)
                    u = plsc.load_gather(eu, [lane, dvec])
                    v = plsc.load_gather(ev, [lane, dvec])
                    new = []
                    for n in range(NEG):
                        nn = plsc.load_gather(en, [lane20 + n, dvec])
                        new.append(negs[n] + u * nn)
                    return (pos + u * v, tuple(new))
                init = (zeros, tuple(zeros for _ in range(NEG)))
                return lax.fori_loop(0, EMB_DIM, dbody, init)

            pos_l, negs_l = side(eul, evl, enl)
            pos_r, negs_r = side(eur, evr, enr)

            sco[0, :] = -pos_l
            sco[1, :] = -pos_r
            for n in range(NEG):
                sco[2 + n, :] = negs_l[n]
                sco[2 + NEG + n, :] = negs_r[n]

            pltpu.sync_copy(sco, out_h.at[wid * NSUB + j])
            return carry

        lax.fori_loop(0, NSUB, sub_body, 0)

    return k(u_l, u_r, v_l, v_r, pu2, pvl2, pvr2, nl2, nr2)


def _tc_reduce(scores):
    def red(x_ref, o_ref):
        x = x_ref[...]
        s = jnp.clip(x, -10.0, 10.0)
        v = jnp.maximum(s, 0.0) + jnp.log(1.0 + jnp.exp(-jnp.abs(s)))
        col = lax.broadcasted_iota(jnp.int32, x.shape, 1)
        v = jnp.where(col < (2 + 2 * NEG) * SUB, v, 0.0)
        o_ref[0, 0] = jnp.sum(v) * (1.0 / BATCH)

    out = pl.pallas_call(
        red,
        out_shape=jax.ShapeDtypeStruct((1, 1), jnp.float32),
        out_specs=pl.BlockSpec(memory_space=pltpu.SMEM),
    )(scores)
    return out[0, 0]


def kernel(pos_u, pos_v_l, pos_v_r, neg_v_l, neg_v_r,
           u_l_weight, u_r_weight, v_l_weight, v_r_weight):
    pu = pos_u.astype(jnp.int32).reshape(BATCH // SUB, SUB)
    pvl = pos_v_l.astype(jnp.int32).reshape(BATCH // SUB, SUB)
    pvr = pos_v_r.astype(jnp.int32).reshape(BATCH // SUB, SUB)
    nl = neg_v_l.astype(jnp.int32).reshape(-1, PART)
    nr = neg_v_r.astype(jnp.int32).reshape(-1, PART)
    scores = _sc_scores(u_l_weight, u_r_weight, v_l_weight, v_r_weight,
                        pu, pvl, pvr, nl, nr)
    return _tc_reduce(scores.reshape(BATCH // SUB, OUT_COLS * SUB))
